# baseline (speedup 1.0000x reference)
import numpy as np

import jax
import jax.numpy as jnp
from jax import lax
from jax.experimental import pallas as pl
from jax.experimental.pallas import tpu as pltpu
from jax.experimental.pallas import tpu_sc as plsc

N = 10000
DEG = 16
DIN = 256
DOUT = 256
L = 16
NW = 32
G = 8
BLKS0 = 64
BLKS1 = 16
BMAX = max(BLKS0, BLKS1)
NBLK = 16 * (BLKS0 + BLKS1)
NPAD = NBLK * G

_LO_IDX = np.empty((128,), np.int32)
_HI_IDX = np.empty((128,), np.int32)
for _c in range(DOUT // 32):
    for _i in range(16):
        _LO_IDX[16 * _c + _i] = 32 * _c + _i
        _HI_IDX[16 * _c + _i] = 32 * _c + 16 + _i


def _tc_body(x_ref, wlo_ref, whi_ref, blo_ref, bhi_ref, alo_ref, ahi_ref,
             c_ref, hw_ref, s2_ref):
    x = x_ref[...]
    hlo = jnp.dot(x, wlo_ref[...], preferred_element_type=jnp.float32) + blo_ref[...]
    hhi = jnp.dot(x, whi_ref[...], preferred_element_type=jnp.float32) + bhi_ref[...]
    s2_ref[...] = (jnp.dot(hlo, alo_ref[...], preferred_element_type=jnp.float32)
                   + jnp.dot(hhi, ahi_ref[...], preferred_element_type=jnp.float32)
                   + c_ref[...])
    lo16 = lax.bitcast_convert_type(hlo.astype(jnp.bfloat16), jnp.uint16)
    hi16 = lax.bitcast_convert_type(hhi.astype(jnp.bfloat16), jnp.uint16)
    w = lo16.astype(jnp.uint32) | (hi16.astype(jnp.uint32) << 16)
    hw_ref[...] = lax.bitcast_convert_type(w, jnp.int32)


def _tc_stage(x, Wlo, Whi, blo, bhi, Alo, Ahi, c):
    nb = 10
    rows = N // nb
    return pl.pallas_call(
        _tc_body,
        grid=(nb,),
        in_specs=[
            pl.BlockSpec((rows, DIN), lambda i: (i, 0)),
            pl.BlockSpec((DIN, 128), lambda i: (0, 0)),
            pl.BlockSpec((DIN, 128), lambda i: (0, 0)),
            pl.BlockSpec((1, 128), lambda i: (0, 0)),
            pl.BlockSpec((1, 128), lambda i: (0, 0)),
            pl.BlockSpec((128, 128), lambda i: (0, 0)),
            pl.BlockSpec((128, 128), lambda i: (0, 0)),
            pl.BlockSpec((1, 128), lambda i: (0, 0)),
        ],
        out_specs=[
            pl.BlockSpec((rows, 128), lambda i: (i, 0)),
            pl.BlockSpec((rows, 128), lambda i: (i, 0)),
        ],
        out_shape=[
            jax.ShapeDtypeStruct((N, 128), jnp.int32),
            jax.ShapeDtypeStruct((N, 128), jnp.float32),
        ],
    )(x, Wlo, Whi, blo, bhi, Alo, Ahi, c)


def _sc_node(g, blk, nbrs_ref, sself_ref, snbr_ref, rows_ref, out_ref):
    idx = nbrs_ref[blk, pl.ds(g * L, L)]
    sg = plsc.load_gather(snbr_ref, [idx])
    su = sself_ref[pl.ds(blk * G + g, L)][0]
    x = sg + su
    e = jnp.where(x >= 0.0, x, x * jnp.float32(0.01))
    m = jnp.max(e)
    ex = jnp.exp(e - m)
    z = jnp.sum(ex)
    alpha = ex / lax.broadcast_in_dim(z, (L,), ())
    nchunk = DOUT // 32
    acc = [jnp.zeros((L,), jnp.float32) for _ in range(DOUT // L)]
    himask = lax.broadcast_in_dim(jnp.int32(-65536), (L,), ())
    shamt = lax.broadcast_in_dim(jnp.int32(16), (L,), ())
    for k in range(DEG):
        ak = alpha[k]
        row = g * DEG + k
        for c in range(nchunk):
            w = rows_ref[row, pl.ds(c * L, L)]
            lo = plsc.bitcast(lax.shift_left(w, shamt), jnp.float32)
            hi = plsc.bitcast(lax.bitwise_and(w, himask), jnp.float32)
            acc[2 * c] = acc[2 * c] + ak * lo
            acc[2 * c + 1] = acc[2 * c + 1] + ak * hi
    for j in range(DOUT // L):
        out_ref[g, pl.ds(j * L, L)] = acc[j]


def _sc_body(h_hbm, snbr_hbm, sself_hbm, nbrs_hbm, out_hbm,
             snbr_v, sself_v, nbrs_v, rows_v, out_v, gsem0, gsem1, osem0, osem1):
    c = lax.axis_index("c")
    s = lax.axis_index("s")
    bbase = s * (BLKS0 + BLKS1) + c * BLKS0
    nblk = BLKS0 + c * (BLKS1 - BLKS0)
    base = bbase * G
    gsems = (gsem0, gsem1)
    osems = (osem0, osem1)
    pltpu.sync_copy(snbr_hbm, snbr_v)
    pltpu.sync_copy(sself_hbm.at[pl.ds(base, G * BMAX + L)], sself_v)
    pltpu.sync_copy(nbrs_hbm.at[pl.ds(bbase, BMAX)], nbrs_v)

    def start_gather(blk, buf):
        pltpu.make_async_copy(
            h_hbm.at[nbrs_v.at[blk]], rows_v.at[buf], gsems[buf]).start()

    def wait_gather(blk, buf):
        pltpu.make_async_copy(
            h_hbm.at[nbrs_v.at[blk]], rows_v.at[buf], gsems[buf]).wait()

    def start_out(blk, buf):
        pltpu.make_async_copy(
            out_v.at[buf], out_hbm.at[pl.ds(base + blk * G, G)], osems[buf]).start()

    def wait_out(blk, buf):
        pltpu.make_async_copy(
            out_v.at[buf], out_hbm.at[pl.ds(base + blk * G, G)], osems[buf]).wait()

    start_gather(0, 0)

    def pair_body(i2, carry):
        for b in range(2):
            blk = i2 * 2 + b

            @pl.when(blk + 1 < nblk)
            def _():
                start_gather(blk + 1, 1 - b)

            wait_gather(blk, b)

            @pl.when(blk >= 2)
            def _():
                wait_out(blk - 2, b)

            def g_body(g, c2):
                _sc_node(g, blk, nbrs_v, sself_v, snbr_v, rows_v.at[b], out_v.at[b])
                return c2

            lax.fori_loop(0, G, g_body, 0)
            start_out(blk, b)
        return carry

    lax.fori_loop(0, nblk // 2, pair_body, 0)
    wait_out(nblk - 2, 0)
    wait_out(nblk - 1, 1)


def _sc_stage(hb, s_nbr, sself_w, nbrs_w):
    mesh = plsc.VectorSubcoreMesh(core_axis_name="c", subcore_axis_name="s")
    fn = pl.kernel(
        _sc_body,
        out_type=jax.ShapeDtypeStruct((NPAD, DOUT), jnp.float32),
        mesh=mesh,
        compiler_params=pltpu.CompilerParams(needs_layout_passes=False),
        scratch_types=[
            pltpu.VMEM((N,), jnp.float32),
            pltpu.VMEM((G * BMAX + L,), jnp.float32),
            pltpu.VMEM((BMAX, G * DEG), jnp.int32),
            pltpu.VMEM((2, G * DEG, 128), jnp.int32),
            pltpu.VMEM((2, G, DOUT), jnp.float32),
            pltpu.SemaphoreType.DMA,
            pltpu.SemaphoreType.DMA,
            pltpu.SemaphoreType.DMA,
            pltpu.SemaphoreType.DMA,
        ],
    )
    return fn(hb, s_nbr, sself_w, nbrs_w)


@jax.jit
def _run(features, neighbors, W, b, a_w, a_b):
    lo_idx = jnp.asarray(_LO_IDX)
    hi_idx = jnp.asarray(_HI_IDX)
    Wlo = W[:, lo_idx]
    Whi = W[:, hi_idx]
    blo = b[lo_idx].reshape(1, 128)
    bhi = b[hi_idx].reshape(1, 128)
    Alo = jnp.zeros((128, 128), jnp.float32)
    Alo = Alo.at[:, 0].set(a_w[:DOUT][lo_idx]).at[:, 1].set(a_w[DOUT:][lo_idx])
    Ahi = jnp.zeros((128, 128), jnp.float32)
    Ahi = Ahi.at[:, 0].set(a_w[:DOUT][hi_idx]).at[:, 1].set(a_w[DOUT:][hi_idx])
    c = jnp.zeros((1, 128), jnp.float32).at[0, 0].set(a_b)
    hw, s2 = _tc_stage(features, Wlo, Whi, blo, bhi, Alo, Ahi, c)
    s_self = s2[:, 0]
    s_nbr = s2[:, 1]
    sself_w = jnp.pad(s_self, (0, NPAD - N + G * BMAX + L))
    nbrs_w = jnp.pad(
        jnp.pad(neighbors, ((0, NPAD - N), (0, 0))).reshape(NBLK, G * DEG),
        ((0, BMAX), (0, 0)))
    out = _sc_stage(hw, s_nbr, sself_w, nbrs_w)
    return out[:N]


def kernel(features, nodes, neighbors, W, b, a_w, a_b):
    del nodes
    return _run(features, neighbors, W, b, a_w, a_b)


# speedup vs baseline: 1.0045x; 1.0045x over previous
import numpy as np

import jax
import jax.numpy as jnp
from jax import lax
from jax.experimental import pallas as pl
from jax.experimental.pallas import tpu as pltpu
from jax.experimental.pallas import tpu_sc as plsc

N = 10000
DEG = 16
DIN = 256
DOUT = 256
L = 16
NW = 32
G = 8
BLKS0 = 56
BLKS1 = 24
BMAX = max(BLKS0, BLKS1)
NBLK = 16 * (BLKS0 + BLKS1)
NPAD = NBLK * G

_LO_IDX = np.empty((128,), np.int32)
_HI_IDX = np.empty((128,), np.int32)
for _c in range(DOUT // 32):
    for _i in range(16):
        _LO_IDX[16 * _c + _i] = 32 * _c + _i
        _HI_IDX[16 * _c + _i] = 32 * _c + 16 + _i


def _tc_body(x_ref, wlo_ref, whi_ref, blo_ref, bhi_ref, alo_ref, ahi_ref,
             c_ref, hw_ref, s2_ref):
    x = x_ref[...]
    hlo = jnp.dot(x, wlo_ref[...], preferred_element_type=jnp.float32) + blo_ref[...]
    hhi = jnp.dot(x, whi_ref[...], preferred_element_type=jnp.float32) + bhi_ref[...]
    s2_ref[...] = (jnp.dot(hlo, alo_ref[...], preferred_element_type=jnp.float32)
                   + jnp.dot(hhi, ahi_ref[...], preferred_element_type=jnp.float32)
                   + c_ref[...])
    lo16 = lax.bitcast_convert_type(hlo.astype(jnp.bfloat16), jnp.uint16)
    hi16 = lax.bitcast_convert_type(hhi.astype(jnp.bfloat16), jnp.uint16)
    w = lo16.astype(jnp.uint32) | (hi16.astype(jnp.uint32) << 16)
    hw_ref[...] = lax.bitcast_convert_type(w, jnp.int32)


def _tc_stage(x, Wlo, Whi, blo, bhi, Alo, Ahi, c):
    nb = 10
    rows = N // nb
    return pl.pallas_call(
        _tc_body,
        grid=(nb,),
        in_specs=[
            pl.BlockSpec((rows, DIN), lambda i: (i, 0)),
            pl.BlockSpec((DIN, 128), lambda i: (0, 0)),
            pl.BlockSpec((DIN, 128), lambda i: (0, 0)),
            pl.BlockSpec((1, 128), lambda i: (0, 0)),
            pl.BlockSpec((1, 128), lambda i: (0, 0)),
            pl.BlockSpec((128, 128), lambda i: (0, 0)),
            pl.BlockSpec((128, 128), lambda i: (0, 0)),
            pl.BlockSpec((1, 128), lambda i: (0, 0)),
        ],
        out_specs=[
            pl.BlockSpec((rows, 128), lambda i: (i, 0)),
            pl.BlockSpec((rows, 128), lambda i: (i, 0)),
        ],
        out_shape=[
            jax.ShapeDtypeStruct((N, 128), jnp.int32),
            jax.ShapeDtypeStruct((N, 128), jnp.float32),
        ],
    )(x, Wlo, Whi, blo, bhi, Alo, Ahi, c)


def _sc_node(g, blk, nbrs_ref, sself_ref, snbr_ref, rows_ref, out_ref):
    idx = nbrs_ref[blk, pl.ds(g * L, L)]
    sg = plsc.load_gather(snbr_ref, [idx])
    su = sself_ref[pl.ds(blk * G + g, L)][0]
    x = sg + su
    e = jnp.where(x >= 0.0, x, x * jnp.float32(0.01))
    m = jnp.max(e)
    ex = jnp.exp(e - m)
    z = jnp.sum(ex)
    alpha = ex / lax.broadcast_in_dim(z, (L,), ())
    nchunk = DOUT // 32
    acc = [jnp.zeros((L,), jnp.float32) for _ in range(DOUT // L)]
    himask = lax.broadcast_in_dim(jnp.int32(-65536), (L,), ())
    shamt = lax.broadcast_in_dim(jnp.int32(16), (L,), ())
    for k in range(DEG):
        ak = alpha[k]
        row = g * DEG + k
        for c in range(nchunk):
            w = rows_ref[row, pl.ds(c * L, L)]
            lo = plsc.bitcast(lax.shift_left(w, shamt), jnp.float32)
            hi = plsc.bitcast(lax.bitwise_and(w, himask), jnp.float32)
            acc[2 * c] = acc[2 * c] + ak * lo
            acc[2 * c + 1] = acc[2 * c + 1] + ak * hi
    for j in range(DOUT // L):
        out_ref[g, pl.ds(j * L, L)] = acc[j]


def _sc_body(h_hbm, snbr_hbm, sself_hbm, nbrs_hbm, out_hbm,
             snbr_v, sself_v, nbrs_v, rows_v, out_v, gsem0, gsem1, osem0, osem1):
    c = lax.axis_index("c")
    s = lax.axis_index("s")
    bbase = s * (BLKS0 + BLKS1) + c * BLKS0
    nblk = BLKS0 + c * (BLKS1 - BLKS0)
    base = bbase * G
    gsems = (gsem0, gsem1)
    osems = (osem0, osem1)
    pltpu.sync_copy(snbr_hbm, snbr_v)
    pltpu.sync_copy(sself_hbm.at[pl.ds(base, G * BMAX + L)], sself_v)
    pltpu.sync_copy(nbrs_hbm.at[pl.ds(bbase, BMAX)], nbrs_v)

    def start_gather(blk, buf):
        pltpu.make_async_copy(
            h_hbm.at[nbrs_v.at[blk]], rows_v.at[buf], gsems[buf]).start()

    def wait_gather(blk, buf):
        pltpu.make_async_copy(
            h_hbm.at[nbrs_v.at[blk]], rows_v.at[buf], gsems[buf]).wait()

    def start_out(blk, buf):
        pltpu.make_async_copy(
            out_v.at[buf], out_hbm.at[pl.ds(base + blk * G, G)], osems[buf]).start()

    def wait_out(blk, buf):
        pltpu.make_async_copy(
            out_v.at[buf], out_hbm.at[pl.ds(base + blk * G, G)], osems[buf]).wait()

    start_gather(0, 0)

    def pair_body(i2, carry):
        for b in range(2):
            blk = i2 * 2 + b

            @pl.when(blk + 1 < nblk)
            def _():
                start_gather(blk + 1, 1 - b)

            wait_gather(blk, b)

            @pl.when(blk >= 2)
            def _():
                wait_out(blk - 2, b)

            def g_body(g, c2):
                _sc_node(g, blk, nbrs_v, sself_v, snbr_v, rows_v.at[b], out_v.at[b])
                return c2

            lax.fori_loop(0, G, g_body, 0)
            start_out(blk, b)
        return carry

    lax.fori_loop(0, nblk // 2, pair_body, 0)
    wait_out(nblk - 2, 0)
    wait_out(nblk - 1, 1)


def _sc_stage(hb, s_nbr, sself_w, nbrs_w):
    mesh = plsc.VectorSubcoreMesh(core_axis_name="c", subcore_axis_name="s")
    fn = pl.kernel(
        _sc_body,
        out_type=jax.ShapeDtypeStruct((NPAD, DOUT), jnp.float32),
        mesh=mesh,
        compiler_params=pltpu.CompilerParams(needs_layout_passes=False),
        scratch_types=[
            pltpu.VMEM((N,), jnp.float32),
            pltpu.VMEM((G * BMAX + L,), jnp.float32),
            pltpu.VMEM((BMAX, G * DEG), jnp.int32),
            pltpu.VMEM((2, G * DEG, 128), jnp.int32),
            pltpu.VMEM((2, G, DOUT), jnp.float32),
            pltpu.SemaphoreType.DMA,
            pltpu.SemaphoreType.DMA,
            pltpu.SemaphoreType.DMA,
            pltpu.SemaphoreType.DMA,
        ],
    )
    return fn(hb, s_nbr, sself_w, nbrs_w)


@jax.jit
def _run(features, neighbors, W, b, a_w, a_b):
    lo_idx = jnp.asarray(_LO_IDX)
    hi_idx = jnp.asarray(_HI_IDX)
    Wlo = W[:, lo_idx]
    Whi = W[:, hi_idx]
    blo = b[lo_idx].reshape(1, 128)
    bhi = b[hi_idx].reshape(1, 128)
    Alo = jnp.zeros((128, 128), jnp.float32)
    Alo = Alo.at[:, 0].set(a_w[:DOUT][lo_idx]).at[:, 1].set(a_w[DOUT:][lo_idx])
    Ahi = jnp.zeros((128, 128), jnp.float32)
    Ahi = Ahi.at[:, 0].set(a_w[:DOUT][hi_idx]).at[:, 1].set(a_w[DOUT:][hi_idx])
    c = jnp.zeros((1, 128), jnp.float32).at[0, 0].set(a_b)
    hw, s2 = _tc_stage(features, Wlo, Whi, blo, bhi, Alo, Ahi, c)
    s_self = s2[:, 0]
    s_nbr = s2[:, 1]
    sself_w = jnp.pad(s_self, (0, NPAD - N + G * BMAX + L))
    nbrs_w = jnp.pad(
        jnp.pad(neighbors, ((0, NPAD - N), (0, 0))).reshape(NBLK, G * DEG),
        ((0, BMAX), (0, 0)))
    out = _sc_stage(hw, s_nbr, sself_w, nbrs_w)
    return out[:N]


def kernel(features, nodes, neighbors, W, b, a_w, a_b):
    del nodes
    return _run(features, neighbors, W, b, a_w, a_b)


# speedup vs baseline: 1.0295x; 1.0249x over previous
import numpy as np

import jax
import jax.numpy as jnp
from jax import lax
from jax.experimental import pallas as pl
from jax.experimental.pallas import tpu as pltpu
from jax.experimental.pallas import tpu_sc as plsc

N = 10000
DEG = 16
DIN = 256
DOUT = 256
L = 16
NW = 32
G = 8
BLKS0 = 56
BLKS1 = 24
BMAX = max(BLKS0, BLKS1)
NBLK = 16 * (BLKS0 + BLKS1)
NPAD = NBLK * G

_LO_IDX = np.empty((128,), np.int32)
_HI_IDX = np.empty((128,), np.int32)
for _c in range(DOUT // 32):
    for _i in range(16):
        _LO_IDX[16 * _c + _i] = 32 * _c + _i
        _HI_IDX[16 * _c + _i] = 32 * _c + 16 + _i


def _tc_body(x_ref, wlo_ref, whi_ref, blo_ref, bhi_ref, alo_ref, ahi_ref,
             c_ref, hw_ref, s2_ref):
    x = x_ref[...]
    hlo = jnp.dot(x, wlo_ref[...], preferred_element_type=jnp.float32) + blo_ref[...]
    hhi = jnp.dot(x, whi_ref[...], preferred_element_type=jnp.float32) + bhi_ref[...]
    s2_ref[...] = (jnp.dot(hlo, alo_ref[...], preferred_element_type=jnp.float32)
                   + jnp.dot(hhi, ahi_ref[...], preferred_element_type=jnp.float32)
                   + c_ref[...])
    lo16 = lax.bitcast_convert_type(hlo.astype(jnp.bfloat16), jnp.uint16)
    hi16 = lax.bitcast_convert_type(hhi.astype(jnp.bfloat16), jnp.uint16)
    w = lo16.astype(jnp.uint32) | (hi16.astype(jnp.uint32) << 16)
    hw_ref[...] = lax.bitcast_convert_type(w, jnp.int32)


def _tc_stage(x, Wlo, Whi, blo, bhi, Alo, Ahi, c):
    nb = 10
    rows = N // nb
    return pl.pallas_call(
        _tc_body,
        grid=(nb,),
        in_specs=[
            pl.BlockSpec((rows, DIN), lambda i: (i, 0)),
            pl.BlockSpec((DIN, 128), lambda i: (0, 0)),
            pl.BlockSpec((DIN, 128), lambda i: (0, 0)),
            pl.BlockSpec((1, 128), lambda i: (0, 0)),
            pl.BlockSpec((1, 128), lambda i: (0, 0)),
            pl.BlockSpec((128, 128), lambda i: (0, 0)),
            pl.BlockSpec((128, 128), lambda i: (0, 0)),
            pl.BlockSpec((1, 128), lambda i: (0, 0)),
        ],
        out_specs=[
            pl.BlockSpec((rows, 128), lambda i: (i, 0)),
            pl.BlockSpec((rows, 128), lambda i: (i, 0)),
        ],
        out_shape=[
            jax.ShapeDtypeStruct((N, 128), jnp.int32),
            jax.ShapeDtypeStruct((N, 128), jnp.float32),
        ],
    )(x, Wlo, Whi, blo, bhi, Alo, Ahi, c)


def _sc_node(g, blk, nbrs_ref, sself_ref, snbr_ref, rows_ref, out_ref):
    idx = nbrs_ref[blk, pl.ds(g * L, L)]
    sg = plsc.load_gather(snbr_ref, [idx])
    su = sself_ref[pl.ds(blk * G + g, L)][0]
    x = sg + su
    e = jnp.where(x >= 0.0, x, x * jnp.float32(0.01))
    m = jnp.max(e)
    ex = jnp.exp(e - m)
    z = jnp.sum(ex)
    alpha = ex / lax.broadcast_in_dim(z, (L,), ())
    nchunk = DOUT // 32
    acc0 = [jnp.zeros((2 * L,), jnp.bfloat16) for _ in range(nchunk)]
    acc1 = [jnp.zeros((2 * L,), jnp.bfloat16) for _ in range(nchunk)]
    for k in range(DEG):
        av = lax.broadcast_in_dim(alpha[k], (L,), ())
        akb = plsc.pack(av, av, format=plsc.PackFormat.INTERLEAVED)
        row = g * DEG + k
        accs = acc0 if k % 2 == 0 else acc1
        for c in range(nchunk):
            w = rows_ref[row, pl.ds(c * L, L)]
            v = plsc.bitcast(w, jnp.bfloat16)
            accs[c] = accs[c] + akb * v
    for c in range(nchunk):
        a0, b0 = plsc.unpack(acc0[c], format=plsc.PackFormat.INTERLEAVED)
        a1, b1 = plsc.unpack(acc1[c], format=plsc.PackFormat.INTERLEAVED)
        out_ref[g, pl.ds(c * 32, L)] = a0 + a1
        out_ref[g, pl.ds(c * 32 + L, L)] = b0 + b1


def _sc_body(h_hbm, snbr_hbm, sself_hbm, nbrs_hbm, out_hbm,
             snbr_v, sself_v, nbrs_v, rows_v, out_v, gsem0, gsem1, osem0, osem1):
    c = lax.axis_index("c")
    s = lax.axis_index("s")
    bbase = s * (BLKS0 + BLKS1) + c * BLKS0
    nblk = BLKS0 + c * (BLKS1 - BLKS0)
    base = bbase * G
    gsems = (gsem0, gsem1)
    osems = (osem0, osem1)
    pltpu.sync_copy(snbr_hbm, snbr_v)
    pltpu.sync_copy(sself_hbm.at[pl.ds(base, G * BMAX + L)], sself_v)
    pltpu.sync_copy(nbrs_hbm.at[pl.ds(bbase, BMAX)], nbrs_v)

    def start_gather(blk, buf):
        pltpu.make_async_copy(
            h_hbm.at[nbrs_v.at[blk]], rows_v.at[buf], gsems[buf]).start()

    def wait_gather(blk, buf):
        pltpu.make_async_copy(
            h_hbm.at[nbrs_v.at[blk]], rows_v.at[buf], gsems[buf]).wait()

    def start_out(blk, buf):
        pltpu.make_async_copy(
            out_v.at[buf], out_hbm.at[pl.ds(base + blk * G, G)], osems[buf]).start()

    def wait_out(blk, buf):
        pltpu.make_async_copy(
            out_v.at[buf], out_hbm.at[pl.ds(base + blk * G, G)], osems[buf]).wait()

    start_gather(0, 0)

    def pair_body(i2, carry):
        for b in range(2):
            blk = i2 * 2 + b

            @pl.when(blk + 1 < nblk)
            def _():
                start_gather(blk + 1, 1 - b)

            wait_gather(blk, b)

            @pl.when(blk >= 2)
            def _():
                wait_out(blk - 2, b)

            def g_body(g, c2):
                _sc_node(g, blk, nbrs_v, sself_v, snbr_v, rows_v.at[b], out_v.at[b])
                return c2

            lax.fori_loop(0, G, g_body, 0)
            start_out(blk, b)
        return carry

    lax.fori_loop(0, nblk // 2, pair_body, 0)
    wait_out(nblk - 2, 0)
    wait_out(nblk - 1, 1)


def _sc_stage(hb, s_nbr, sself_w, nbrs_w):
    mesh = plsc.VectorSubcoreMesh(core_axis_name="c", subcore_axis_name="s")
    fn = pl.kernel(
        _sc_body,
        out_type=jax.ShapeDtypeStruct((NPAD, DOUT), jnp.float32),
        mesh=mesh,
        compiler_params=pltpu.CompilerParams(needs_layout_passes=False),
        scratch_types=[
            pltpu.VMEM((N,), jnp.float32),
            pltpu.VMEM((G * BMAX + L,), jnp.float32),
            pltpu.VMEM((BMAX, G * DEG), jnp.int32),
            pltpu.VMEM((2, G * DEG, 128), jnp.int32),
            pltpu.VMEM((2, G, DOUT), jnp.float32),
            pltpu.SemaphoreType.DMA,
            pltpu.SemaphoreType.DMA,
            pltpu.SemaphoreType.DMA,
            pltpu.SemaphoreType.DMA,
        ],
    )
    return fn(hb, s_nbr, sself_w, nbrs_w)


@jax.jit
def _run(features, neighbors, W, b, a_w, a_b):
    lo_idx = jnp.asarray(_LO_IDX)
    hi_idx = jnp.asarray(_HI_IDX)
    Wlo = W[:, lo_idx]
    Whi = W[:, hi_idx]
    blo = b[lo_idx].reshape(1, 128)
    bhi = b[hi_idx].reshape(1, 128)
    Alo = jnp.zeros((128, 128), jnp.float32)
    Alo = Alo.at[:, 0].set(a_w[:DOUT][lo_idx]).at[:, 1].set(a_w[DOUT:][lo_idx])
    Ahi = jnp.zeros((128, 128), jnp.float32)
    Ahi = Ahi.at[:, 0].set(a_w[:DOUT][hi_idx]).at[:, 1].set(a_w[DOUT:][hi_idx])
    c = jnp.zeros((1, 128), jnp.float32).at[0, 0].set(a_b)
    hw, s2 = _tc_stage(features, Wlo, Whi, blo, bhi, Alo, Ahi, c)
    s_self = s2[:, 0]
    s_nbr = s2[:, 1]
    sself_w = jnp.pad(s_self, (0, NPAD - N + G * BMAX + L))
    nbrs_w = jnp.pad(
        jnp.pad(neighbors, ((0, NPAD - N), (0, 0))).reshape(NBLK, G * DEG),
        ((0, BMAX), (0, 0)))
    out = _sc_stage(hw, s_nbr, sself_w, nbrs_w)
    return out[:N]


def kernel(features, nodes, neighbors, W, b, a_w, a_b):
    del nodes
    return _run(features, neighbors, W, b, a_w, a_b)


# speedup vs baseline: 1.0424x; 1.0125x over previous
import numpy as np

import jax
import jax.numpy as jnp
from jax import lax
from jax.experimental import pallas as pl
from jax.experimental.pallas import tpu as pltpu
from jax.experimental.pallas import tpu_sc as plsc

N = 10000
DEG = 16
DIN = 256
DOUT = 256
L = 16
NW = 32
G = 8
BLKS0 = 64
BLKS1 = 16
BMAX = max(BLKS0, BLKS1)
NBLK = 16 * (BLKS0 + BLKS1)
NPAD = NBLK * G

_LO_IDX = np.empty((128,), np.int32)
_HI_IDX = np.empty((128,), np.int32)
for _c in range(DOUT // 32):
    for _i in range(16):
        _LO_IDX[16 * _c + _i] = 32 * _c + _i
        _HI_IDX[16 * _c + _i] = 32 * _c + 16 + _i


def _tc_body(x_ref, wlo_ref, whi_ref, blo_ref, bhi_ref, alo_ref, ahi_ref,
             c_ref, hw_ref, s2_ref):
    x = x_ref[...]
    hlo = jnp.dot(x, wlo_ref[...], preferred_element_type=jnp.float32) + blo_ref[...]
    hhi = jnp.dot(x, whi_ref[...], preferred_element_type=jnp.float32) + bhi_ref[...]
    s2_ref[...] = (jnp.dot(hlo, alo_ref[...], preferred_element_type=jnp.float32)
                   + jnp.dot(hhi, ahi_ref[...], preferred_element_type=jnp.float32)
                   + c_ref[...])
    lo16 = lax.bitcast_convert_type(hlo.astype(jnp.bfloat16), jnp.uint16)
    hi16 = lax.bitcast_convert_type(hhi.astype(jnp.bfloat16), jnp.uint16)
    w = lo16.astype(jnp.uint32) | (hi16.astype(jnp.uint32) << 16)
    hw_ref[...] = lax.bitcast_convert_type(w, jnp.int32)


def _tc_stage(x, Wlo, Whi, blo, bhi, Alo, Ahi, c):
    nb = 10
    rows = N // nb
    return pl.pallas_call(
        _tc_body,
        grid=(nb,),
        in_specs=[
            pl.BlockSpec((rows, DIN), lambda i: (i, 0)),
            pl.BlockSpec((DIN, 128), lambda i: (0, 0)),
            pl.BlockSpec((DIN, 128), lambda i: (0, 0)),
            pl.BlockSpec((1, 128), lambda i: (0, 0)),
            pl.BlockSpec((1, 128), lambda i: (0, 0)),
            pl.BlockSpec((128, 128), lambda i: (0, 0)),
            pl.BlockSpec((128, 128), lambda i: (0, 0)),
            pl.BlockSpec((1, 128), lambda i: (0, 0)),
        ],
        out_specs=[
            pl.BlockSpec((rows, 128), lambda i: (i, 0)),
            pl.BlockSpec((rows, 128), lambda i: (i, 0)),
        ],
        out_shape=[
            jax.ShapeDtypeStruct((N, 128), jnp.int32),
            jax.ShapeDtypeStruct((N, 128), jnp.float32),
        ],
    )(x, Wlo, Whi, blo, bhi, Alo, Ahi, c)


def _sc_node(g, blk, nbrs_ref, sself_ref, snbr_ref, rows_ref, out_ref):
    idx = nbrs_ref[blk, pl.ds(g * L, L)]
    sg = plsc.load_gather(snbr_ref, [idx])
    su = sself_ref[pl.ds(blk * G + g, L)][0]
    x = sg + su
    e = jnp.where(x >= 0.0, x, x * jnp.float32(0.01))
    m = jnp.max(e)
    ex = jnp.exp(e - m)
    z = jnp.sum(ex)
    alpha = ex / lax.broadcast_in_dim(z, (L,), ())
    nchunk = DOUT // 32
    acc0 = [jnp.zeros((2 * L,), jnp.bfloat16) for _ in range(nchunk)]
    acc1 = [jnp.zeros((2 * L,), jnp.bfloat16) for _ in range(nchunk)]
    for k in range(DEG):
        av = lax.broadcast_in_dim(alpha[k], (L,), ())
        akb = plsc.pack(av, av, format=plsc.PackFormat.INTERLEAVED)
        row = g * DEG + k
        accs = acc0 if k % 2 == 0 else acc1
        for c in range(nchunk):
            w = rows_ref[row, pl.ds(c * L, L)]
            v = plsc.bitcast(w, jnp.bfloat16)
            accs[c] = accs[c] + akb * v
    for c in range(nchunk):
        a0, b0 = plsc.unpack(acc0[c], format=plsc.PackFormat.INTERLEAVED)
        a1, b1 = plsc.unpack(acc1[c], format=plsc.PackFormat.INTERLEAVED)
        out_ref[g, pl.ds(c * 32, L)] = a0 + a1
        out_ref[g, pl.ds(c * 32 + L, L)] = b0 + b1


def _sc_body(h_hbm, snbr_hbm, sself_hbm, nbrs_hbm, out_hbm,
             snbr_v, sself_v, nbrs_v, rows_v, out_v, gsem0, gsem1, osem0, osem1):
    c = lax.axis_index("c")
    s = lax.axis_index("s")
    bbase = s * (BLKS0 + BLKS1) + c * BLKS0
    nblk = BLKS0 + c * (BLKS1 - BLKS0)
    base = bbase * G
    gsems = (gsem0, gsem1)
    osems = (osem0, osem1)
    pltpu.sync_copy(snbr_hbm, snbr_v)
    pltpu.sync_copy(sself_hbm.at[pl.ds(base, G * BMAX + L)], sself_v)
    pltpu.sync_copy(nbrs_hbm.at[pl.ds(bbase, BMAX)], nbrs_v)

    def start_gather(blk, buf):
        pltpu.make_async_copy(
            h_hbm.at[nbrs_v.at[blk]], rows_v.at[buf], gsems[buf]).start()

    def wait_gather(blk, buf):
        pltpu.make_async_copy(
            h_hbm.at[nbrs_v.at[blk]], rows_v.at[buf], gsems[buf]).wait()

    def start_out(blk, buf):
        pltpu.make_async_copy(
            out_v.at[buf], out_hbm.at[pl.ds(base + blk * G, G)], osems[buf]).start()

    def wait_out(blk, buf):
        pltpu.make_async_copy(
            out_v.at[buf], out_hbm.at[pl.ds(base + blk * G, G)], osems[buf]).wait()

    start_gather(0, 0)

    def pair_body(i2, carry):
        for b in range(2):
            blk = i2 * 2 + b

            @pl.when(blk + 1 < nblk)
            def _():
                start_gather(blk + 1, 1 - b)

            wait_gather(blk, b)

            @pl.when(blk >= 2)
            def _():
                wait_out(blk - 2, b)

            def g_body(g, c2):
                _sc_node(g, blk, nbrs_v, sself_v, snbr_v, rows_v.at[b], out_v.at[b])
                return c2

            lax.fori_loop(0, G, g_body, 0)
            start_out(blk, b)
        return carry

    lax.fori_loop(0, nblk // 2, pair_body, 0)
    wait_out(nblk - 2, 0)
    wait_out(nblk - 1, 1)


def _sc_stage(hb, s_nbr, sself_w, nbrs_w):
    mesh = plsc.VectorSubcoreMesh(core_axis_name="c", subcore_axis_name="s")
    fn = pl.kernel(
        _sc_body,
        out_type=jax.ShapeDtypeStruct((NPAD, DOUT), jnp.float32),
        mesh=mesh,
        compiler_params=pltpu.CompilerParams(needs_layout_passes=False),
        scratch_types=[
            pltpu.VMEM((N,), jnp.float32),
            pltpu.VMEM((G * BMAX + L,), jnp.float32),
            pltpu.VMEM((BMAX, G * DEG), jnp.int32),
            pltpu.VMEM((2, G * DEG, 128), jnp.int32),
            pltpu.VMEM((2, G, DOUT), jnp.float32),
            pltpu.SemaphoreType.DMA,
            pltpu.SemaphoreType.DMA,
            pltpu.SemaphoreType.DMA,
            pltpu.SemaphoreType.DMA,
        ],
    )
    return fn(hb, s_nbr, sself_w, nbrs_w)


@jax.jit
def _run(features, neighbors, W, b, a_w, a_b):
    lo_idx = jnp.asarray(_LO_IDX)
    hi_idx = jnp.asarray(_HI_IDX)
    Wlo = W[:, lo_idx]
    Whi = W[:, hi_idx]
    blo = b[lo_idx].reshape(1, 128)
    bhi = b[hi_idx].reshape(1, 128)
    Alo = jnp.zeros((128, 128), jnp.float32)
    Alo = Alo.at[:, 0].set(a_w[:DOUT][lo_idx]).at[:, 1].set(a_w[DOUT:][lo_idx])
    Ahi = jnp.zeros((128, 128), jnp.float32)
    Ahi = Ahi.at[:, 0].set(a_w[:DOUT][hi_idx]).at[:, 1].set(a_w[DOUT:][hi_idx])
    c = jnp.zeros((1, 128), jnp.float32).at[0, 0].set(a_b)
    hw, s2 = _tc_stage(features, Wlo, Whi, blo, bhi, Alo, Ahi, c)
    s_self = s2[:, 0]
    s_nbr = s2[:, 1]
    sself_w = jnp.pad(s_self, (0, NPAD - N + G * BMAX + L))
    nbrs_w = jnp.pad(
        jnp.pad(neighbors, ((0, NPAD - N), (0, 0))).reshape(NBLK, G * DEG),
        ((0, BMAX), (0, 0)))
    out = _sc_stage(hw, s_nbr, sself_w, nbrs_w)
    return out[:N]


def kernel(features, nodes, neighbors, W, b, a_w, a_b):
    del nodes
    return _run(features, neighbors, W, b, a_w, a_b)


# speedup vs baseline: 1.0881x; 1.0439x over previous
import numpy as np

import jax
import jax.numpy as jnp
from jax import lax
from jax.experimental import pallas as pl
from jax.experimental.pallas import tpu as pltpu
from jax.experimental.pallas import tpu_sc as plsc

N = 10000
DEG = 16
DIN = 256
DOUT = 256
L = 16
NW = 32
G = 8
BLKS0 = 64
BLKS1 = 16
BMAX = max(BLKS0, BLKS1)
NBLK = 16 * (BLKS0 + BLKS1)
NPAD = NBLK * G

_LO_IDX = np.empty((128,), np.int32)
_HI_IDX = np.empty((128,), np.int32)
for _c in range(DOUT // 32):
    for _i in range(16):
        _LO_IDX[16 * _c + _i] = 32 * _c + _i
        _HI_IDX[16 * _c + _i] = 32 * _c + 16 + _i


def _tc_body(x_ref, wlo_ref, whi_ref, blo_ref, bhi_ref, alo_ref, ahi_ref,
             c_ref, hw_ref, s2_ref):
    x = x_ref[...]
    hlo = jnp.dot(x, wlo_ref[...], preferred_element_type=jnp.float32) + blo_ref[...]
    hhi = jnp.dot(x, whi_ref[...], preferred_element_type=jnp.float32) + bhi_ref[...]
    s2_ref[...] = (jnp.dot(hlo, alo_ref[...], preferred_element_type=jnp.float32)
                   + jnp.dot(hhi, ahi_ref[...], preferred_element_type=jnp.float32)
                   + c_ref[...])
    lo16 = lax.bitcast_convert_type(hlo.astype(jnp.bfloat16), jnp.uint16)
    hi16 = lax.bitcast_convert_type(hhi.astype(jnp.bfloat16), jnp.uint16)
    w = lo16.astype(jnp.uint32) | (hi16.astype(jnp.uint32) << 16)
    hw_ref[...] = lax.bitcast_convert_type(w, jnp.int32)


def _tc_stage(x, Wlo, Whi, blo, bhi, Alo, Ahi, c):
    nb = 10
    rows = N // nb
    return pl.pallas_call(
        _tc_body,
        grid=(nb,),
        in_specs=[
            pl.BlockSpec((rows, DIN), lambda i: (i, 0)),
            pl.BlockSpec((DIN, 128), lambda i: (0, 0)),
            pl.BlockSpec((DIN, 128), lambda i: (0, 0)),
            pl.BlockSpec((1, 128), lambda i: (0, 0)),
            pl.BlockSpec((1, 128), lambda i: (0, 0)),
            pl.BlockSpec((128, 128), lambda i: (0, 0)),
            pl.BlockSpec((128, 128), lambda i: (0, 0)),
            pl.BlockSpec((1, 128), lambda i: (0, 0)),
        ],
        out_specs=[
            pl.BlockSpec((rows, 128), lambda i: (i, 0)),
            pl.BlockSpec((rows, 128), lambda i: (i, 0)),
        ],
        out_shape=[
            jax.ShapeDtypeStruct((N, 128), jnp.int32),
            jax.ShapeDtypeStruct((N, 128), jnp.float32),
        ],
    )(x, Wlo, Whi, blo, bhi, Alo, Ahi, c)


def _sc_node(g, blk, nbrs_ref, sself_ref, snbr_ref, rows_ref, out_ref):
    idx = nbrs_ref[blk, pl.ds(g * L, L)]
    sg = plsc.load_gather(snbr_ref, [idx])
    su = sself_ref[pl.ds(blk * G + g, L)][0]
    x = sg + su
    e = jnp.where(x >= 0.0, x, x * jnp.float32(0.01))
    m = jnp.max(e)
    ex = jnp.exp(e - m)
    z = jnp.sum(ex)
    alpha = ex / lax.broadcast_in_dim(z, (L,), ())
    nchunk = DOUT // 32
    acc0 = [jnp.zeros((2 * L,), jnp.bfloat16) for _ in range(nchunk)]
    acc1 = [jnp.zeros((2 * L,), jnp.bfloat16) for _ in range(nchunk)]
    for k in range(DEG):
        av = lax.broadcast_in_dim(alpha[k], (L,), ())
        akb = plsc.pack(av, av, format=plsc.PackFormat.INTERLEAVED)
        row = g * DEG + k
        accs = acc0 if k % 2 == 0 else acc1
        for c in range(nchunk):
            w = rows_ref[row, pl.ds(c * L, L)]
            v = plsc.bitcast(w, jnp.bfloat16)
            accs[c] = accs[c] + akb * v
    for c in range(nchunk):
        a0, b0 = plsc.unpack(acc0[c], format=plsc.PackFormat.INTERLEAVED)
        a1, b1 = plsc.unpack(acc1[c], format=plsc.PackFormat.INTERLEAVED)
        out_ref[g, pl.ds(c * 32, L)] = a0 + a1
        out_ref[g, pl.ds(c * 32 + L, L)] = b0 + b1


def _sc_body(h_hbm, snbr_hbm, sself_hbm, nbrs_hbm, out_hbm,
             snbr_v, sself_v, nbrs_v, rows_v, out_v,
             gsem0, gsem1, gsem2, gsem3, osem0, osem1):
    c = lax.axis_index("c")
    s = lax.axis_index("s")
    bbase = s * (BLKS0 + BLKS1) + c * BLKS0
    nblk = BLKS0 + c * (BLKS1 - BLKS0)
    base = bbase * G
    gsems = (gsem0, gsem1, gsem2, gsem3)
    osems = (osem0, osem1)
    pltpu.sync_copy(snbr_hbm, snbr_v)
    pltpu.sync_copy(sself_hbm.at[pl.ds(base, G * BMAX + L)], sself_v)
    pltpu.sync_copy(nbrs_hbm.at[pl.ds(bbase, BMAX)], nbrs_v)

    def start_gather(blk, buf):
        pltpu.make_async_copy(
            h_hbm.at[nbrs_v.at[blk]], rows_v.at[buf], gsems[buf]).start()

    def wait_gather(blk, buf):
        pltpu.make_async_copy(
            h_hbm.at[nbrs_v.at[blk]], rows_v.at[buf], gsems[buf]).wait()

    def start_out(blk, buf):
        pltpu.make_async_copy(
            out_v.at[buf], out_hbm.at[pl.ds(base + blk * G, G)], osems[buf]).start()

    def wait_out(blk, buf):
        pltpu.make_async_copy(
            out_v.at[buf], out_hbm.at[pl.ds(base + blk * G, G)], osems[buf]).wait()

    for j in range(3):
        start_gather(j, j)

    def quad_body(i4, carry):
        for b in range(4):
            blk = i4 * 4 + b
            ob = b % 2

            @pl.when(blk + 3 < nblk)
            def _():
                start_gather(blk + 3, (b + 3) % 4)

            wait_gather(blk, b)

            @pl.when(blk >= 2)
            def _():
                wait_out(blk - 2, ob)

            def g_body(g, c2):
                _sc_node(g, blk, nbrs_v, sself_v, snbr_v, rows_v.at[b], out_v.at[ob])
                return c2

            lax.fori_loop(0, G, g_body, 0)
            start_out(blk, ob)
        return carry

    lax.fori_loop(0, nblk // 4, quad_body, 0)
    wait_out(nblk - 2, 0)
    wait_out(nblk - 1, 1)


def _sc_stage(hb, s_nbr, sself_w, nbrs_w):
    mesh = plsc.VectorSubcoreMesh(core_axis_name="c", subcore_axis_name="s")
    fn = pl.kernel(
        _sc_body,
        out_type=jax.ShapeDtypeStruct((NPAD, DOUT), jnp.float32),
        mesh=mesh,
        compiler_params=pltpu.CompilerParams(needs_layout_passes=False),
        scratch_types=[
            pltpu.VMEM((N,), jnp.float32),
            pltpu.VMEM((G * BMAX + L,), jnp.float32),
            pltpu.VMEM((BMAX, G * DEG), jnp.int32),
            pltpu.VMEM((4, G * DEG, 128), jnp.int32),
            pltpu.VMEM((2, G, DOUT), jnp.float32),
            pltpu.SemaphoreType.DMA,
            pltpu.SemaphoreType.DMA,
            pltpu.SemaphoreType.DMA,
            pltpu.SemaphoreType.DMA,
            pltpu.SemaphoreType.DMA,
            pltpu.SemaphoreType.DMA,
        ],
    )
    return fn(hb, s_nbr, sself_w, nbrs_w)


@jax.jit
def _run(features, neighbors, W, b, a_w, a_b):
    lo_idx = jnp.asarray(_LO_IDX)
    hi_idx = jnp.asarray(_HI_IDX)
    Wlo = W[:, lo_idx]
    Whi = W[:, hi_idx]
    blo = b[lo_idx].reshape(1, 128)
    bhi = b[hi_idx].reshape(1, 128)
    Alo = jnp.zeros((128, 128), jnp.float32)
    Alo = Alo.at[:, 0].set(a_w[:DOUT][lo_idx]).at[:, 1].set(a_w[DOUT:][lo_idx])
    Ahi = jnp.zeros((128, 128), jnp.float32)
    Ahi = Ahi.at[:, 0].set(a_w[:DOUT][hi_idx]).at[:, 1].set(a_w[DOUT:][hi_idx])
    c = jnp.zeros((1, 128), jnp.float32).at[0, 0].set(a_b)
    hw, s2 = _tc_stage(features, Wlo, Whi, blo, bhi, Alo, Ahi, c)
    s_self = s2[:, 0]
    s_nbr = s2[:, 1]
    sself_w = jnp.pad(s_self, (0, NPAD - N + G * BMAX + L))
    nbrs_w = jnp.pad(
        jnp.pad(neighbors, ((0, NPAD - N), (0, 0))).reshape(NBLK, G * DEG),
        ((0, BMAX), (0, 0)))
    out = _sc_stage(hw, s_nbr, sself_w, nbrs_w)
    return out[:N]


def kernel(features, nodes, neighbors, W, b, a_w, a_b):
    del nodes
    return _run(features, neighbors, W, b, a_w, a_b)


# speedup vs baseline: 1.1155x; 1.0252x over previous
import numpy as np

import jax
import jax.numpy as jnp
from jax import lax
from jax.experimental import pallas as pl
from jax.experimental.pallas import tpu as pltpu
from jax.experimental.pallas import tpu_sc as plsc

N = 10000
DEG = 16
DIN = 256
DOUT = 256
L = 16
NW = 32
G = 8
BLKS0 = 64
BLKS1 = 16
BMAX = max(BLKS0, BLKS1)
NBLK = 16 * (BLKS0 + BLKS1)
NPAD = NBLK * G

_LO_IDX = np.empty((128,), np.int32)
_HI_IDX = np.empty((128,), np.int32)
for _c in range(DOUT // 32):
    for _i in range(16):
        _LO_IDX[16 * _c + _i] = 32 * _c + _i
        _HI_IDX[16 * _c + _i] = 32 * _c + 16 + _i


def _tc_body(x_ref, wlo_ref, whi_ref, blo_ref, bhi_ref, alo_ref, ahi_ref,
             c_ref, hw_ref, s2_ref):
    x = x_ref[...]
    hlo = jnp.dot(x, wlo_ref[...], preferred_element_type=jnp.float32) + blo_ref[...]
    hhi = jnp.dot(x, whi_ref[...], preferred_element_type=jnp.float32) + bhi_ref[...]
    s2_ref[...] = (jnp.dot(hlo, alo_ref[...], preferred_element_type=jnp.float32)
                   + jnp.dot(hhi, ahi_ref[...], preferred_element_type=jnp.float32)
                   + c_ref[...])
    lo16 = lax.bitcast_convert_type(hlo.astype(jnp.bfloat16), jnp.uint16)
    hi16 = lax.bitcast_convert_type(hhi.astype(jnp.bfloat16), jnp.uint16)
    w = lo16.astype(jnp.uint32) | (hi16.astype(jnp.uint32) << 16)
    hw_ref[...] = lax.bitcast_convert_type(w, jnp.int32)


def _tc_stage(x, Wlo, Whi, blo, bhi, Alo, Ahi, c):
    nb = 10
    rows = N // nb
    return pl.pallas_call(
        _tc_body,
        grid=(nb,),
        in_specs=[
            pl.BlockSpec((rows, DIN), lambda i: (i, 0)),
            pl.BlockSpec((DIN, 128), lambda i: (0, 0)),
            pl.BlockSpec((DIN, 128), lambda i: (0, 0)),
            pl.BlockSpec((1, 128), lambda i: (0, 0)),
            pl.BlockSpec((1, 128), lambda i: (0, 0)),
            pl.BlockSpec((128, 128), lambda i: (0, 0)),
            pl.BlockSpec((128, 128), lambda i: (0, 0)),
            pl.BlockSpec((1, 128), lambda i: (0, 0)),
        ],
        out_specs=[
            pl.BlockSpec((rows, 128), lambda i: (i, 0)),
            pl.BlockSpec((rows, 128), lambda i: (i, 0)),
        ],
        out_shape=[
            jax.ShapeDtypeStruct((N, 128), jnp.int32),
            jax.ShapeDtypeStruct((N, 128), jnp.float32),
        ],
    )(x, Wlo, Whi, blo, bhi, Alo, Ahi, c)


def _sc_node(g, blk, nbrs_ref, sself_ref, snbr_ref, rows_ref, out_ref):
    idx = nbrs_ref[blk, pl.ds(g * L, L)]
    sg = plsc.load_gather(snbr_ref, [idx])
    su = sself_ref[pl.ds(blk * G + g, L)][0]
    x = sg + su
    e = jnp.where(x >= 0.0, x, x * jnp.float32(0.01))
    m = jnp.max(e)
    ex = jnp.exp(e - m)
    z = jnp.sum(ex)
    alpha = ex / lax.broadcast_in_dim(z, (L,), ())
    nchunk = DOUT // 32
    acc0 = [jnp.zeros((2 * L,), jnp.bfloat16) for _ in range(nchunk)]
    acc1 = [jnp.zeros((2 * L,), jnp.bfloat16) for _ in range(nchunk)]
    for k in range(DEG):
        av = lax.broadcast_in_dim(alpha[k], (L,), ())
        akb = plsc.pack(av, av, format=plsc.PackFormat.INTERLEAVED)
        row = g * DEG + k
        accs = acc0 if k % 2 == 0 else acc1
        for c in range(nchunk):
            w = rows_ref[row, pl.ds(c * L, L)]
            v = plsc.bitcast(w, jnp.bfloat16)
            accs[c] = accs[c] + akb * v
    for c in range(nchunk):
        a0, b0 = plsc.unpack(acc0[c], format=plsc.PackFormat.INTERLEAVED)
        a1, b1 = plsc.unpack(acc1[c], format=plsc.PackFormat.INTERLEAVED)
        out_ref[g, pl.ds(c * 32, L)] = a0 + a1
        out_ref[g, pl.ds(c * 32 + L, L)] = b0 + b1


def _sc_body(h_hbm, snbr_hbm, sself_hbm, nbrs_hbm, out_hbm, junk_hbm,
             snbr_v, sself_v, nbrs_v, rows_v, out_v,
             gsem0, gsem1, gsem2, gsem3, osem0, osem1):
    c = lax.axis_index("c")
    s = lax.axis_index("s")
    bbase = s * (BLKS0 + BLKS1) + c * BLKS0
    nblk = BLKS0 + c * (BLKS1 - BLKS0)
    base = bbase * G
    gsems = (gsem0, gsem1, gsem2, gsem3)
    osems = (osem0, osem1)
    pltpu.sync_copy(snbr_hbm, snbr_v)
    pltpu.sync_copy(sself_hbm.at[pl.ds(base, G * BMAX + L)], sself_v)
    pltpu.sync_copy(nbrs_hbm.at[pl.ds(bbase, BMAX)], nbrs_v)

    def start_gather(blk, buf):
        pltpu.make_async_copy(
            h_hbm.at[nbrs_v.at[blk]], rows_v.at[buf], gsems[buf]).start()

    def wait_gather(blk, buf):
        pltpu.make_async_copy(
            h_hbm.at[nbrs_v.at[blk]], rows_v.at[buf], gsems[buf]).wait()

    def start_out(blk, buf):
        row = base + blk * G

        @pl.when(row < N)
        def _():
            pltpu.make_async_copy(
                out_v.at[buf], out_hbm.at[pl.ds(row, G)], osems[buf]).start()

        @pl.when(row >= N)
        def _():
            pltpu.make_async_copy(
                out_v.at[buf], junk_hbm.at[pl.ds(0, G)], osems[buf]).start()

    def wait_out(blk, buf):
        row = base + blk * G

        @pl.when(row < N)
        def _():
            pltpu.make_async_copy(
                out_v.at[buf], out_hbm.at[pl.ds(row, G)], osems[buf]).wait()

        @pl.when(row >= N)
        def _():
            pltpu.make_async_copy(
                out_v.at[buf], junk_hbm.at[pl.ds(0, G)], osems[buf]).wait()

    for j in range(3):
        start_gather(j, j)

    def quad_body(i4, carry):
        for b in range(4):
            blk = i4 * 4 + b
            ob = b % 2

            @pl.when(blk + 3 < nblk)
            def _():
                start_gather(blk + 3, (b + 3) % 4)

            wait_gather(blk, b)

            @pl.when(blk >= 2)
            def _():
                wait_out(blk - 2, ob)

            def g_body(g, c2):
                _sc_node(g, blk, nbrs_v, sself_v, snbr_v, rows_v.at[b], out_v.at[ob])
                return c2

            lax.fori_loop(0, G, g_body, 0)
            start_out(blk, ob)
        return carry

    lax.fori_loop(0, nblk // 4, quad_body, 0)
    wait_out(nblk - 2, 0)
    wait_out(nblk - 1, 1)


def _sc_stage(hb, s_nbr, sself_w, nbrs_w):
    mesh = plsc.VectorSubcoreMesh(core_axis_name="c", subcore_axis_name="s")
    fn = pl.kernel(
        _sc_body,
        out_type=[jax.ShapeDtypeStruct((N, DOUT), jnp.float32),
                  jax.ShapeDtypeStruct((G, DOUT), jnp.float32)],
        mesh=mesh,
        compiler_params=pltpu.CompilerParams(needs_layout_passes=False),
        scratch_types=[
            pltpu.VMEM((N,), jnp.float32),
            pltpu.VMEM((G * BMAX + L,), jnp.float32),
            pltpu.VMEM((BMAX, G * DEG), jnp.int32),
            pltpu.VMEM((4, G * DEG, 128), jnp.int32),
            pltpu.VMEM((2, G, DOUT), jnp.float32),
            pltpu.SemaphoreType.DMA,
            pltpu.SemaphoreType.DMA,
            pltpu.SemaphoreType.DMA,
            pltpu.SemaphoreType.DMA,
            pltpu.SemaphoreType.DMA,
            pltpu.SemaphoreType.DMA,
        ],
    )
    return fn(hb, s_nbr, sself_w, nbrs_w)[0]


@jax.jit
def _run(features, neighbors, W, b, a_w, a_b):
    lo_idx = jnp.asarray(_LO_IDX)
    hi_idx = jnp.asarray(_HI_IDX)
    Wlo = W[:, lo_idx]
    Whi = W[:, hi_idx]
    blo = b[lo_idx].reshape(1, 128)
    bhi = b[hi_idx].reshape(1, 128)
    Alo = jnp.zeros((128, 128), jnp.float32)
    Alo = Alo.at[:, 0].set(a_w[:DOUT][lo_idx]).at[:, 1].set(a_w[DOUT:][lo_idx])
    Ahi = jnp.zeros((128, 128), jnp.float32)
    Ahi = Ahi.at[:, 0].set(a_w[:DOUT][hi_idx]).at[:, 1].set(a_w[DOUT:][hi_idx])
    c = jnp.zeros((1, 128), jnp.float32).at[0, 0].set(a_b)
    hw, s2 = _tc_stage(features, Wlo, Whi, blo, bhi, Alo, Ahi, c)
    s_self = s2[:, 0]
    s_nbr = s2[:, 1]
    sself_w = jnp.pad(s_self, (0, NPAD - N + G * BMAX + L))
    nbrs_w = jnp.pad(
        jnp.pad(neighbors, ((0, NPAD - N), (0, 0))).reshape(NBLK, G * DEG),
        ((0, BMAX), (0, 0)))
    return _sc_stage(hw, s_nbr, sself_w, nbrs_w)


def kernel(features, nodes, neighbors, W, b, a_w, a_b):
    del nodes
    return _run(features, neighbors, W, b, a_w, a_b)


# speedup vs baseline: 1.1194x; 1.0035x over previous
import numpy as np

import jax
import jax.numpy as jnp
from jax import lax
from jax.experimental import pallas as pl
from jax.experimental.pallas import tpu as pltpu
from jax.experimental.pallas import tpu_sc as plsc

N = 10000
DEG = 16
DIN = 256
DOUT = 256
L = 16
NW = 32
G = 8
BLKS0 = 64
BLKS1 = 16
BMAX = max(BLKS0, BLKS1)
NBLK = 16 * (BLKS0 + BLKS1)
NPAD = NBLK * G

_LO_IDX = np.empty((128,), np.int32)
_HI_IDX = np.empty((128,), np.int32)
for _c in range(DOUT // 32):
    for _i in range(16):
        _LO_IDX[16 * _c + _i] = 32 * _c + _i
        _HI_IDX[16 * _c + _i] = 32 * _c + 16 + _i


def _tc_body(x_ref, wlo_ref, whi_ref, blo_ref, bhi_ref, alo_ref, ahi_ref,
             c_ref, hw_ref, s2_ref):
    x = x_ref[...]
    hlo = jnp.dot(x, wlo_ref[...], preferred_element_type=jnp.float32) + blo_ref[...]
    hhi = jnp.dot(x, whi_ref[...], preferred_element_type=jnp.float32) + bhi_ref[...]
    s2_ref[...] = (jnp.dot(hlo, alo_ref[...], preferred_element_type=jnp.float32)
                   + jnp.dot(hhi, ahi_ref[...], preferred_element_type=jnp.float32)
                   + c_ref[...])
    lo16 = lax.bitcast_convert_type(hlo.astype(jnp.bfloat16), jnp.uint16)
    hi16 = lax.bitcast_convert_type(hhi.astype(jnp.bfloat16), jnp.uint16)
    w = lo16.astype(jnp.uint32) | (hi16.astype(jnp.uint32) << 16)
    hw_ref[...] = lax.bitcast_convert_type(w, jnp.int32)


def _tc_stage(x, Wlo, Whi, blo, bhi, Alo, Ahi, c):
    nb = 10
    rows = N // nb
    return pl.pallas_call(
        _tc_body,
        grid=(nb,),
        in_specs=[
            pl.BlockSpec((rows, DIN), lambda i: (i, 0)),
            pl.BlockSpec((DIN, 128), lambda i: (0, 0)),
            pl.BlockSpec((DIN, 128), lambda i: (0, 0)),
            pl.BlockSpec((1, 128), lambda i: (0, 0)),
            pl.BlockSpec((1, 128), lambda i: (0, 0)),
            pl.BlockSpec((128, 128), lambda i: (0, 0)),
            pl.BlockSpec((128, 128), lambda i: (0, 0)),
            pl.BlockSpec((1, 128), lambda i: (0, 0)),
        ],
        out_specs=[
            pl.BlockSpec((rows, 128), lambda i: (i, 0)),
            pl.BlockSpec((rows, 128), lambda i: (i, 0)),
        ],
        out_shape=[
            jax.ShapeDtypeStruct((N, 128), jnp.int32),
            jax.ShapeDtypeStruct((N, 128), jnp.float32),
        ],
    )(x, Wlo, Whi, blo, bhi, Alo, Ahi, c)


def _sc_node(g, blk, nbrs_ref, sself_ref, snbr_ref, rows_ref, out_ref):
    idx = nbrs_ref[blk, pl.ds(g * L, L)]
    sg = plsc.load_gather(snbr_ref, [idx])
    su = sself_ref[pl.ds(blk * G + g, L)][0]
    x = sg + su
    e = jnp.where(x >= 0.0, x, x * jnp.float32(0.01))
    m = jnp.max(e)
    ex = jnp.exp(e - m)
    z = jnp.sum(ex)
    alpha = ex / lax.broadcast_in_dim(z, (L,), ())
    nchunk = DOUT // 32
    acc0 = [jnp.zeros((2 * L,), jnp.bfloat16) for _ in range(nchunk)]
    acc1 = [jnp.zeros((2 * L,), jnp.bfloat16) for _ in range(nchunk)]
    for k in range(DEG):
        av = lax.broadcast_in_dim(alpha[k], (L,), ())
        akb = plsc.pack(av, av, format=plsc.PackFormat.INTERLEAVED)
        row = g * DEG + k
        accs = acc0 if k % 2 == 0 else acc1
        for c in range(nchunk):
            w = rows_ref[row, pl.ds(c * L, L)]
            v = plsc.bitcast(w, jnp.bfloat16)
            accs[c] = accs[c] + akb * v
    for c in range(nchunk):
        a0, b0 = plsc.unpack(acc0[c], format=plsc.PackFormat.INTERLEAVED)
        a1, b1 = plsc.unpack(acc1[c], format=plsc.PackFormat.INTERLEAVED)
        out_ref[g, pl.ds(c * 32, L)] = a0 + a1
        out_ref[g, pl.ds(c * 32 + L, L)] = b0 + b1


def _sc_body(h_hbm, snbr_hbm, sself_hbm, nbrs_hbm, out_hbm, junk_hbm,
             snbr_v, sself_v, nbrs_v, rows_v, out_v,
             gsem0, gsem1, gsem2, gsem3, osem0, osem1):
    c = lax.axis_index("c")
    s = lax.axis_index("s")
    bbase = s * (BLKS0 + BLKS1) + c * BLKS0
    nblk = BLKS0 + c * (BLKS1 - BLKS0)
    base = bbase * G
    gsems = (gsem0, gsem1, gsem2, gsem3)
    osems = (osem0, osem1)
    pltpu.sync_copy(nbrs_hbm.at[pl.ds(bbase, BMAX)], nbrs_v)

    def start_gather(blk, buf):
        pltpu.make_async_copy(
            h_hbm.at[nbrs_v.at[blk]], rows_v.at[buf], gsems[buf]).start()

    def wait_gather(blk, buf):
        pltpu.make_async_copy(
            h_hbm.at[nbrs_v.at[blk]], rows_v.at[buf], gsems[buf]).wait()

    def start_out(blk, buf):
        row = base + blk * G

        @pl.when(row < N)
        def _():
            pltpu.make_async_copy(
                out_v.at[buf], out_hbm.at[pl.ds(row, G)], osems[buf]).start()

        @pl.when(row >= N)
        def _():
            pltpu.make_async_copy(
                out_v.at[buf], junk_hbm.at[pl.ds(0, G)], osems[buf]).start()

    def wait_out(blk, buf):
        row = base + blk * G

        @pl.when(row < N)
        def _():
            pltpu.make_async_copy(
                out_v.at[buf], out_hbm.at[pl.ds(row, G)], osems[buf]).wait()

        @pl.when(row >= N)
        def _():
            pltpu.make_async_copy(
                out_v.at[buf], junk_hbm.at[pl.ds(0, G)], osems[buf]).wait()

    for j in range(3):
        start_gather(j, j)
    pltpu.sync_copy(snbr_hbm, snbr_v)
    pltpu.sync_copy(sself_hbm.at[pl.ds(base, G * BMAX + L)], sself_v)

    def quad_body(i4, carry):
        for b in range(4):
            blk = i4 * 4 + b
            ob = b % 2

            @pl.when(blk + 3 < nblk)
            def _():
                start_gather(blk + 3, (b + 3) % 4)

            wait_gather(blk, b)

            @pl.when(blk >= 2)
            def _():
                wait_out(blk - 2, ob)

            def g_body(g2, c2):
                _sc_node(2 * g2, blk, nbrs_v, sself_v, snbr_v, rows_v.at[b], out_v.at[ob])
                _sc_node(2 * g2 + 1, blk, nbrs_v, sself_v, snbr_v, rows_v.at[b], out_v.at[ob])
                return c2

            lax.fori_loop(0, G // 2, g_body, 0)
            start_out(blk, ob)
        return carry

    lax.fori_loop(0, nblk // 4, quad_body, 0)
    wait_out(nblk - 2, 0)
    wait_out(nblk - 1, 1)


def _sc_stage(hb, s_nbr, sself_w, nbrs_w):
    mesh = plsc.VectorSubcoreMesh(core_axis_name="c", subcore_axis_name="s")
    fn = pl.kernel(
        _sc_body,
        out_type=[jax.ShapeDtypeStruct((N, DOUT), jnp.float32),
                  jax.ShapeDtypeStruct((G, DOUT), jnp.float32)],
        mesh=mesh,
        compiler_params=pltpu.CompilerParams(needs_layout_passes=False),
        scratch_types=[
            pltpu.VMEM((N,), jnp.float32),
            pltpu.VMEM((G * BMAX + L,), jnp.float32),
            pltpu.VMEM((BMAX, G * DEG), jnp.int32),
            pltpu.VMEM((4, G * DEG, 128), jnp.int32),
            pltpu.VMEM((2, G, DOUT), jnp.float32),
            pltpu.SemaphoreType.DMA,
            pltpu.SemaphoreType.DMA,
            pltpu.SemaphoreType.DMA,
            pltpu.SemaphoreType.DMA,
            pltpu.SemaphoreType.DMA,
            pltpu.SemaphoreType.DMA,
        ],
    )
    return fn(hb, s_nbr, sself_w, nbrs_w)[0]


@jax.jit
def _run(features, neighbors, W, b, a_w, a_b):
    lo_idx = jnp.asarray(_LO_IDX)
    hi_idx = jnp.asarray(_HI_IDX)
    Wlo = W[:, lo_idx]
    Whi = W[:, hi_idx]
    blo = b[lo_idx].reshape(1, 128)
    bhi = b[hi_idx].reshape(1, 128)
    Alo = jnp.zeros((128, 128), jnp.float32)
    Alo = Alo.at[:, 0].set(a_w[:DOUT][lo_idx]).at[:, 1].set(a_w[DOUT:][lo_idx])
    Ahi = jnp.zeros((128, 128), jnp.float32)
    Ahi = Ahi.at[:, 0].set(a_w[:DOUT][hi_idx]).at[:, 1].set(a_w[DOUT:][hi_idx])
    c = jnp.zeros((1, 128), jnp.float32).at[0, 0].set(a_b)
    hw, s2 = _tc_stage(features, Wlo, Whi, blo, bhi, Alo, Ahi, c)
    s_self = s2[:, 0]
    s_nbr = s2[:, 1]
    sself_w = jnp.pad(s_self, (0, NPAD - N + G * BMAX + L))
    nbrs_w = jnp.pad(
        jnp.pad(neighbors, ((0, NPAD - N), (0, 0))).reshape(NBLK, G * DEG),
        ((0, BMAX), (0, 0)))
    return _sc_stage(hw, s_nbr, sself_w, nbrs_w)


def kernel(features, nodes, neighbors, W, b, a_w, a_b):
    del nodes
    return _run(features, neighbors, W, b, a_w, a_b)


# speedup vs baseline: 1.1216x; 1.0019x over previous
import numpy as np

import jax
import jax.numpy as jnp
from jax import lax
from jax.experimental import pallas as pl
from jax.experimental.pallas import tpu as pltpu
from jax.experimental.pallas import tpu_sc as plsc

N = 10000
DEG = 16
DIN = 256
DOUT = 256
L = 16
NW = 32
G = 8
BLKS0 = 64
BLKS1 = 16
BMAX = max(BLKS0, BLKS1)
NBLK = 16 * (BLKS0 + BLKS1)
NPAD = NBLK * G

_LO_IDX = np.empty((128,), np.int32)
_HI_IDX = np.empty((128,), np.int32)
for _c in range(DOUT // 32):
    for _i in range(16):
        _LO_IDX[16 * _c + _i] = 32 * _c + _i
        _HI_IDX[16 * _c + _i] = 32 * _c + 16 + _i


def _tc_body(x_ref, wlo_ref, whi_ref, blo_ref, bhi_ref, alo_ref, ahi_ref,
             c_ref, hw_ref, s2_ref):
    x = x_ref[...]
    hlo = jnp.dot(x, wlo_ref[...], preferred_element_type=jnp.float32) + blo_ref[...]
    hhi = jnp.dot(x, whi_ref[...], preferred_element_type=jnp.float32) + bhi_ref[...]
    s2_ref[...] = (jnp.dot(hlo, alo_ref[...], preferred_element_type=jnp.float32)
                   + jnp.dot(hhi, ahi_ref[...], preferred_element_type=jnp.float32)
                   + c_ref[...])
    lo16 = lax.bitcast_convert_type(hlo.astype(jnp.bfloat16), jnp.uint16)
    hi16 = lax.bitcast_convert_type(hhi.astype(jnp.bfloat16), jnp.uint16)
    w = lo16.astype(jnp.uint32) | (hi16.astype(jnp.uint32) << 16)
    hw_ref[...] = lax.bitcast_convert_type(w, jnp.int32)


def _tc_stage(x, Wlo, Whi, blo, bhi, Alo, Ahi, c):
    nb = 10
    rows = N // nb
    return pl.pallas_call(
        _tc_body,
        grid=(nb,),
        in_specs=[
            pl.BlockSpec((rows, DIN), lambda i: (i, 0)),
            pl.BlockSpec((DIN, 128), lambda i: (0, 0)),
            pl.BlockSpec((DIN, 128), lambda i: (0, 0)),
            pl.BlockSpec((1, 128), lambda i: (0, 0)),
            pl.BlockSpec((1, 128), lambda i: (0, 0)),
            pl.BlockSpec((128, 128), lambda i: (0, 0)),
            pl.BlockSpec((128, 128), lambda i: (0, 0)),
            pl.BlockSpec((1, 128), lambda i: (0, 0)),
        ],
        out_specs=[
            pl.BlockSpec((rows, 128), lambda i: (i, 0)),
            pl.BlockSpec((rows, 128), lambda i: (i, 0)),
        ],
        out_shape=[
            jax.ShapeDtypeStruct((N, 128), jnp.int32),
            jax.ShapeDtypeStruct((N, 128), jnp.float32),
        ],
    )(x, Wlo, Whi, blo, bhi, Alo, Ahi, c)


def _sc_node(g, blk, nbrs_ref, sself_ref, snbr_ref, rows_ref, out_ref):
    idx = nbrs_ref[blk, pl.ds(g * L, L)]
    sg = plsc.load_gather(snbr_ref, [idx])
    su = sself_ref[pl.ds(blk * G + g, L)][0]
    x = sg + su
    e = jnp.where(x >= 0.0, x, x * jnp.float32(0.01))
    m = jnp.max(e)
    ex = jnp.exp(e - m)
    z = jnp.sum(ex)
    alpha = ex / lax.broadcast_in_dim(z, (L,), ())
    nchunk = DOUT // 32
    acc0 = [jnp.zeros((2 * L,), jnp.bfloat16) for _ in range(nchunk)]
    acc1 = [jnp.zeros((2 * L,), jnp.bfloat16) for _ in range(nchunk)]
    for k in range(DEG):
        av = lax.broadcast_in_dim(alpha[k], (L,), ())
        akb = plsc.pack(av, av, format=plsc.PackFormat.INTERLEAVED)
        row = g * DEG + k
        accs = acc0 if k % 2 == 0 else acc1
        for c in range(nchunk):
            w = rows_ref[row, pl.ds(c * L, L)]
            v = plsc.bitcast(w, jnp.bfloat16)
            accs[c] = accs[c] + akb * v
    for c in range(nchunk):
        a0, b0 = plsc.unpack(acc0[c], format=plsc.PackFormat.INTERLEAVED)
        a1, b1 = plsc.unpack(acc1[c], format=plsc.PackFormat.INTERLEAVED)
        out_ref[g, pl.ds(c * 32, L)] = a0 + a1
        out_ref[g, pl.ds(c * 32 + L, L)] = b0 + b1


def _sc_body(h_hbm, snbr_hbm, sself_hbm, nbrs_hbm, out_hbm, junk_hbm,
             snbr_sh, snbr_v, sself_v, nbrs_v, rows_v, out_v,
             gsem0, gsem1, gsem2, gsem3, osem0, osem1):
    c = lax.axis_index("c")
    s = lax.axis_index("s")
    bbase = s * (BLKS0 + BLKS1) + c * BLKS0
    nblk = BLKS0 + c * (BLKS1 - BLKS0)
    base = bbase * G
    gsems = (gsem0, gsem1, gsem2, gsem3)
    osems = (osem0, osem1)
    pltpu.sync_copy(nbrs_hbm.at[pl.ds(bbase, BMAX)], nbrs_v)

    def start_gather(blk, buf):
        pltpu.make_async_copy(
            h_hbm.at[nbrs_v.at[blk]], rows_v.at[buf], gsems[buf]).start()

    def wait_gather(blk, buf):
        pltpu.make_async_copy(
            h_hbm.at[nbrs_v.at[blk]], rows_v.at[buf], gsems[buf]).wait()

    def start_out(blk, buf):
        row = base + blk * G

        @pl.when(row < N)
        def _():
            pltpu.make_async_copy(
                out_v.at[buf], out_hbm.at[pl.ds(row, G)], osems[buf]).start()

        @pl.when(row >= N)
        def _():
            pltpu.make_async_copy(
                out_v.at[buf], junk_hbm.at[pl.ds(0, G)], osems[buf]).start()

    def wait_out(blk, buf):
        row = base + blk * G

        @pl.when(row < N)
        def _():
            pltpu.make_async_copy(
                out_v.at[buf], out_hbm.at[pl.ds(row, G)], osems[buf]).wait()

        @pl.when(row >= N)
        def _():
            pltpu.make_async_copy(
                out_v.at[buf], junk_hbm.at[pl.ds(0, G)], osems[buf]).wait()

    for j in range(3):
        start_gather(j, j)

    @pl.when(s == 0)
    def _():
        pltpu.sync_copy(snbr_hbm, snbr_sh)

    pltpu.sync_copy(sself_hbm.at[pl.ds(base, G * BMAX + L)], sself_v)
    plsc.subcore_barrier()
    pltpu.sync_copy(snbr_sh, snbr_v)

    def quad_body(i4, carry):
        for b in range(4):
            blk = i4 * 4 + b
            ob = b % 2

            @pl.when(blk + 3 < nblk)
            def _():
                start_gather(blk + 3, (b + 3) % 4)

            wait_gather(blk, b)

            @pl.when(blk >= 2)
            def _():
                wait_out(blk - 2, ob)

            def g_body(g2, c2):
                _sc_node(2 * g2, blk, nbrs_v, sself_v, snbr_v, rows_v.at[b], out_v.at[ob])
                _sc_node(2 * g2 + 1, blk, nbrs_v, sself_v, snbr_v, rows_v.at[b], out_v.at[ob])
                return c2

            lax.fori_loop(0, G // 2, g_body, 0)
            start_out(blk, ob)
        return carry

    lax.fori_loop(0, nblk // 4, quad_body, 0)
    wait_out(nblk - 2, 0)
    wait_out(nblk - 1, 1)


def _sc_stage(hb, s_nbr, sself_w, nbrs_w):
    mesh = plsc.VectorSubcoreMesh(core_axis_name="c", subcore_axis_name="s")
    fn = pl.kernel(
        _sc_body,
        out_type=[jax.ShapeDtypeStruct((N, DOUT), jnp.float32),
                  jax.ShapeDtypeStruct((G, DOUT), jnp.float32)],
        mesh=mesh,
        compiler_params=pltpu.CompilerParams(needs_layout_passes=False),
        scratch_types=[
            pltpu.VMEM_SHARED((N,), jnp.float32),
            pltpu.VMEM((N,), jnp.float32),
            pltpu.VMEM((G * BMAX + L,), jnp.float32),
            pltpu.VMEM((BMAX, G * DEG), jnp.int32),
            pltpu.VMEM((4, G * DEG, 128), jnp.int32),
            pltpu.VMEM((2, G, DOUT), jnp.float32),
            pltpu.SemaphoreType.DMA,
            pltpu.SemaphoreType.DMA,
            pltpu.SemaphoreType.DMA,
            pltpu.SemaphoreType.DMA,
            pltpu.SemaphoreType.DMA,
            pltpu.SemaphoreType.DMA,
        ],
    )
    return fn(hb, s_nbr, sself_w, nbrs_w)[0]


@jax.jit
def _run(features, neighbors, W, b, a_w, a_b):
    lo_idx = jnp.asarray(_LO_IDX)
    hi_idx = jnp.asarray(_HI_IDX)
    Wlo = W[:, lo_idx]
    Whi = W[:, hi_idx]
    blo = b[lo_idx].reshape(1, 128)
    bhi = b[hi_idx].reshape(1, 128)
    Alo = jnp.zeros((128, 128), jnp.float32)
    Alo = Alo.at[:, 0].set(a_w[:DOUT][lo_idx]).at[:, 1].set(a_w[DOUT:][lo_idx])
    Ahi = jnp.zeros((128, 128), jnp.float32)
    Ahi = Ahi.at[:, 0].set(a_w[:DOUT][hi_idx]).at[:, 1].set(a_w[DOUT:][hi_idx])
    c = jnp.zeros((1, 128), jnp.float32).at[0, 0].set(a_b)
    hw, s2 = _tc_stage(features, Wlo, Whi, blo, bhi, Alo, Ahi, c)
    s_self = s2[:, 0]
    s_nbr = s2[:, 1]
    sself_w = jnp.pad(s_self, (0, NPAD - N + G * BMAX + L))
    nbrs_w = jnp.pad(
        jnp.pad(neighbors, ((0, NPAD - N), (0, 0))).reshape(NBLK, G * DEG),
        ((0, BMAX), (0, 0)))
    return _sc_stage(hw, s_nbr, sself_w, nbrs_w)


def kernel(features, nodes, neighbors, W, b, a_w, a_b):
    del nodes
    return _run(features, neighbors, W, b, a_w, a_b)
